# baseline (speedup 1.0000x reference)
import functools

import jax
import jax.numpy as jnp
from jax import lax
from jax.experimental import pallas as pl
from jax.experimental.pallas import tpu as pltpu
from jax.experimental.pallas import tpu_sc as plsc

_B = 16384
_D = 64
_NPAIR = 5
_NW = 32
_PER_W = _B // _NW
_C = 128
_NCHUNK = _PER_W // _C
_G = _C // 16
_WIDE = 2 * _D
_N = 1000000
_TBLK = 512
_TGRID = (_N + _TBLK - 1) // _TBLK
_NROW = _TGRID * _TBLK // 2


def _tc_detile_kernel(in_ref, out_ref):
    x = in_ref[...]
    t = x.T
    h = _TBLK // 2
    out_ref[...] = jnp.concatenate([t[:h], t[h:]], axis=1)


def _detile(tableT):
    return pl.pallas_call(
        _tc_detile_kernel,
        grid=(_TGRID,),
        in_specs=[pl.BlockSpec((_D, _TBLK), lambda k: (0, k))],
        out_specs=pl.BlockSpec((_TBLK // 2, _WIDE), lambda k: (k, 0)),
        out_shape=jax.ShapeDtypeStruct((_NROW, _WIDE), jnp.float32),
    )(tableT)


def _sc_predictions_kernel(user_table, item_table, uid_hbm, iidT_hbm, out_hbm,
                           uids_v, iids_v, ucol_v, icol_v,
                           u_slab, i_slab, pred_v, sem):
    wid = lax.axis_index("s") * 2 + lax.axis_index("c")
    lane = lax.iota(jnp.int32, 16)

    for c in range(_NCHUNK):
        base = wid * _PER_W + c * _C
        pltpu.sync_copy(uid_hbm.at[pl.ds(base, _C)], uids_v)
        for j in range(_NPAIR):
            pltpu.sync_copy(iidT_hbm.at[pl.ds(j * _B + base, _C)],
                            iids_v.at[pl.ds(j * _C, _C)])
        for k in range(_C // 16):
            v = uids_v[pl.ds(k * 16, 16)]
            ucol_v[pl.ds(k * 16, 16)] = (lax.shift_right_logical(v, 8) & 1) * _D
            uids_v[pl.ds(k * 16, 16)] = (
                lax.shift_right_logical(v, 9) * (_TBLK // 2) + (v & 255))
        for k in range(_NPAIR * _C // 16):
            v = iids_v[pl.ds(k * 16, 16)]
            icol_v[pl.ds(k * 16, 16)] = (lax.shift_right_logical(v, 8) & 1) * _D
            iids_v[pl.ds(k * 16, 16)] = (
                lax.shift_right_logical(v, 9) * (_TBLK // 2) + (v & 255))
        cps = [pltpu.async_copy(user_table.at[uids_v], u_slab, sem)]
        for j in range(_NPAIR):
            cps.append(pltpu.async_copy(
                item_table.at[iids_v.at[pl.ds(j * _C, _C)]],
                i_slab.at[pl.ds(j * _C, _C)], sem))
        for cp in cps:
            cp.wait()

        def group_body(g, carry):
            b0 = g * 16
            urows = b0 + lane
            ucol = ucol_v[pl.ds(b0, 16)]
            accs = [jnp.zeros((16,), jnp.float32) for _ in range(_NPAIR)]
            irows = [j * _C + b0 + lane for j in range(_NPAIR)]
            icols = [icol_v[pl.ds(j * _C + b0, 16)] for j in range(_NPAIR)]
            for d in range(_D):
                uv = plsc.load_gather(u_slab, [urows, ucol + d])
                for j in range(_NPAIR):
                    iv = plsc.load_gather(i_slab, [irows[j], icols[j] + d])
                    accs[j] = accs[j] + uv * iv
            for j in range(_NPAIR):
                pred_v[pl.ds(j * _C + b0, 16)] = accs[j]
            return carry

        lax.fori_loop(0, _G, group_body, 0)
        pltpu.sync_copy(
            pred_v,
            out_hbm.at[pl.ds((wid * _NCHUNK + c) * _NPAIR * _C, _NPAIR * _C)])


def _tc_loss_kernel(pred_ref, out_ref):
    p = pred_ref[...]
    pos = p[:, 0:1, :]
    negs = p[:, 1:_NPAIR, :]
    out_ref[...] = jnp.mean(jax.nn.softplus(negs - pos)).reshape(1, 1)


def kernel(user_table, item_table, user_id, item_id):
    uid = user_id.reshape(_B)
    iidT = item_id.T.reshape(_NPAIR * _B)
    ut = _detile(user_table.T)
    it = _detile(item_table.T)

    sc = pl.kernel(
        _sc_predictions_kernel,
        out_type=jax.ShapeDtypeStruct((_NW * _NCHUNK * _NPAIR * _C,),
                                      jnp.float32),
        mesh=plsc.VectorSubcoreMesh(core_axis_name="c", subcore_axis_name="s"),
        compiler_params=pltpu.CompilerParams(
            needs_layout_passes=False, use_tc_tiling_on_sc=False),
        scratch_types=[
            pltpu.VMEM((_C,), jnp.int32),
            pltpu.VMEM((_NPAIR * _C,), jnp.int32),
            pltpu.VMEM((_C,), jnp.int32),
            pltpu.VMEM((_NPAIR * _C,), jnp.int32),
            pltpu.VMEM((_C, _WIDE), jnp.float32),
            pltpu.VMEM((_NPAIR * _C, _WIDE), jnp.float32),
            pltpu.VMEM((_NPAIR * _C,), jnp.float32),
            pltpu.SemaphoreType.DMA,
        ],
    )
    preds = sc(ut, it, uid, iidT)
    preds = preds.reshape(_NW * _NCHUNK, _NPAIR, _C)

    loss = pl.pallas_call(
        _tc_loss_kernel,
        out_shape=jax.ShapeDtypeStruct((1, 1), jnp.float32),
    )(preds)
    return loss[0, 0]


# speedup vs baseline: 2.3880x; 2.3880x over previous
import functools

import jax
import jax.numpy as jnp
from jax import lax
from jax.experimental import pallas as pl
from jax.experimental.pallas import tpu as pltpu
from jax.experimental.pallas import tpu_sc as plsc

_B = 16384
_D = 64
_NPAIR = 5
_NW = 32
_PER_W = _B // _NW
_C = 128
_NCHUNK = _PER_W // _C
_G = _C // 16
_WIDE = 2 * _D
_N = 1000000
_TBLK = 2048
_TGRID = (_N + _TBLK - 1) // _TBLK
_NROW = _TGRID * _TBLK // 2


def _tc_detile_kernel(in_ref, out_ref):
    x = in_ref[...]
    eye = jnp.eye(_D, dtype=jnp.float32)
    t = jax.lax.dot_general(x, eye, (((0,), (0,)), ((), ())),
                            preferred_element_type=jnp.float32)
    h = _TBLK // 2
    out_ref[...] = jnp.concatenate([t[:h], t[h:]], axis=1)


def _detile(tableT):
    return pl.pallas_call(
        _tc_detile_kernel,
        grid=(_TGRID,),
        in_specs=[pl.BlockSpec((_D, _TBLK), lambda k: (0, k))],
        out_specs=pl.BlockSpec((_TBLK // 2, _WIDE), lambda k: (k, 0)),
        out_shape=jax.ShapeDtypeStruct((_NROW, _WIDE), jnp.float32),
    )(tableT)


def _sc_predictions_kernel(user_table, item_table, uid_hbm, iidT_hbm, out_hbm,
                           uids_v, iids_v, ucol_v, icol_v,
                           u_slab, i_slab, pred_v, sem):
    wid = lax.axis_index("s") * 2 + lax.axis_index("c")
    lane = lax.iota(jnp.int32, 16)

    for c in range(_NCHUNK):
        base = wid * _PER_W + c * _C
        pltpu.sync_copy(uid_hbm.at[pl.ds(base, _C)], uids_v)
        for j in range(_NPAIR):
            pltpu.sync_copy(iidT_hbm.at[pl.ds(j * _B + base, _C)],
                            iids_v.at[pl.ds(j * _C, _C)])
        sh = _TBLK.bit_length() - 1
        hmask = _TBLK // 2 - 1
        for k in range(_C // 16):
            v = uids_v[pl.ds(k * 16, 16)]
            ucol_v[pl.ds(k * 16, 16)] = (
                lax.shift_right_logical(v, sh - 1) & 1) * _D
            uids_v[pl.ds(k * 16, 16)] = (
                lax.shift_right_logical(v, sh) * (_TBLK // 2) + (v & hmask))
        for k in range(_NPAIR * _C // 16):
            v = iids_v[pl.ds(k * 16, 16)]
            icol_v[pl.ds(k * 16, 16)] = (
                lax.shift_right_logical(v, sh - 1) & 1) * _D
            iids_v[pl.ds(k * 16, 16)] = (
                lax.shift_right_logical(v, sh) * (_TBLK // 2) + (v & hmask))
        cps = [pltpu.async_copy(user_table.at[uids_v], u_slab, sem)]
        for j in range(_NPAIR):
            cps.append(pltpu.async_copy(
                item_table.at[iids_v.at[pl.ds(j * _C, _C)]],
                i_slab.at[pl.ds(j * _C, _C)], sem))
        for cp in cps:
            cp.wait()

        def group_body(g, carry):
            b0 = g * 16
            urows = b0 + lane
            ucol = ucol_v[pl.ds(b0, 16)]
            accs = [jnp.zeros((16,), jnp.float32) for _ in range(_NPAIR)]
            irows = [j * _C + b0 + lane for j in range(_NPAIR)]
            icols = [icol_v[pl.ds(j * _C + b0, 16)] for j in range(_NPAIR)]
            for d in range(_D):
                uv = plsc.load_gather(u_slab, [urows, ucol + d])
                for j in range(_NPAIR):
                    iv = plsc.load_gather(i_slab, [irows[j], icols[j] + d])
                    accs[j] = accs[j] + uv * iv
            for j in range(_NPAIR):
                pred_v[pl.ds(j * _C + b0, 16)] = accs[j]
            return carry

        lax.fori_loop(0, _G, group_body, 0)
        pltpu.sync_copy(
            pred_v,
            out_hbm.at[pl.ds((wid * _NCHUNK + c) * _NPAIR * _C, _NPAIR * _C)])


def _tc_loss_kernel(pred_ref, out_ref):
    p = pred_ref[...]
    pos = p[:, 0:1, :]
    negs = p[:, 1:_NPAIR, :]
    out_ref[...] = jnp.mean(jax.nn.softplus(negs - pos)).reshape(1, 1)


def kernel(user_table, item_table, user_id, item_id):
    uid = user_id.reshape(_B)
    iidT = item_id.T.reshape(_NPAIR * _B)
    ut = _detile(user_table.T)
    it = _detile(item_table.T)

    sc = pl.kernel(
        _sc_predictions_kernel,
        out_type=jax.ShapeDtypeStruct((_NW * _NCHUNK * _NPAIR * _C,),
                                      jnp.float32),
        mesh=plsc.VectorSubcoreMesh(core_axis_name="c", subcore_axis_name="s"),
        compiler_params=pltpu.CompilerParams(
            needs_layout_passes=False, use_tc_tiling_on_sc=False),
        scratch_types=[
            pltpu.VMEM((_C,), jnp.int32),
            pltpu.VMEM((_NPAIR * _C,), jnp.int32),
            pltpu.VMEM((_C,), jnp.int32),
            pltpu.VMEM((_NPAIR * _C,), jnp.int32),
            pltpu.VMEM((_C, _WIDE), jnp.float32),
            pltpu.VMEM((_NPAIR * _C, _WIDE), jnp.float32),
            pltpu.VMEM((_NPAIR * _C,), jnp.float32),
            pltpu.SemaphoreType.DMA,
        ],
    )
    preds = sc(ut, it, uid, iidT)
    preds = preds.reshape(_NW * _NCHUNK, _NPAIR, _C)

    loss = pl.pallas_call(
        _tc_loss_kernel,
        out_shape=jax.ShapeDtypeStruct((1, 1), jnp.float32),
    )(preds)
    return loss[0, 0]


# speedup vs baseline: 2.6815x; 1.1229x over previous
import functools

import jax
import jax.numpy as jnp
from jax import lax
from jax.experimental import pallas as pl
from jax.experimental.pallas import tpu as pltpu
from jax.experimental.pallas import tpu_sc as plsc

_B = 16384
_D = 64
_NPAIR = 5
_NW = 32
_PER_W = _B // _NW
_C = 128
_NCHUNK = _PER_W // _C
_G = _C // 16
_WIDE = 2 * _D


def _sc_predictions_kernel(cat_table, uid_hbm, iidT_hbm, out_hbm,
                           uids_v, iids_v, u_slab, i_slab, pred_v, sem):
    wid = lax.axis_index("s") * 2 + lax.axis_index("c")
    lane = lax.iota(jnp.int32, 16)

    for c in range(_NCHUNK):
        base = wid * _PER_W + c * _C
        pltpu.sync_copy(uid_hbm.at[pl.ds(base, _C)], uids_v)
        for j in range(_NPAIR):
            pltpu.sync_copy(iidT_hbm.at[pl.ds(j * _B + base, _C)],
                            iids_v.at[pl.ds(j * _C, _C)])

        def issue_body(g, carry):
            b0 = g * 16
            uvec = uids_v[pl.ds(b0, 16)]
            ivecs = [iids_v[pl.ds(j * _C + b0, 16)] for j in range(_NPAIR)]
            for p in range(16):
                pltpu.async_copy(cat_table.at[pl.ds(uvec[p], 1), :],
                                 u_slab.at[pl.ds(b0 + p, 1), :], sem)
                for j in range(_NPAIR):
                    pltpu.async_copy(
                        cat_table.at[pl.ds(ivecs[j][p], 1), :],
                        i_slab.at[pl.ds(j * _C + b0 + p, 1), :], sem)
            return carry

        lax.fori_loop(0, _G, issue_body, 0)
        pltpu.make_async_copy(cat_table.at[pl.ds(0, _C), :], u_slab,
                              sem).wait()
        pltpu.make_async_copy(cat_table.at[pl.ds(0, _NPAIR * _C), :], i_slab,
                              sem).wait()

        def group_body(g, carry):
            b0 = g * 16
            res = [jnp.zeros((16,), jnp.float32) for _ in range(_NPAIR)]
            for p in range(16):
                b = b0 + p
                us = [u_slab[b, pl.ds(q * 16, 16)] for q in range(_D // 16)]
                for j in range(_NPAIR):
                    r = j * _C + b
                    prod = us[0] * i_slab[r, pl.ds(_D, 16)]
                    for q in range(1, _D // 16):
                        prod = prod + us[q] * i_slab[r, pl.ds(_D + q * 16, 16)]
                    s = jnp.sum(prod)
                    res[j] = jnp.where(lane == p, s, res[j])
            for j in range(_NPAIR):
                pred_v[pl.ds(j * _C + b0, 16)] = res[j]
            return carry

        lax.fori_loop(0, _G, group_body, 0)
        pltpu.sync_copy(
            pred_v,
            out_hbm.at[pl.ds((wid * _NCHUNK + c) * _NPAIR * _C, _NPAIR * _C)])


def _tc_loss_kernel(pred_ref, out_ref):
    p = pred_ref[...]
    pos = p[:, 0:1, :]
    negs = p[:, 1:_NPAIR, :]
    out_ref[...] = jnp.mean(jax.nn.softplus(negs - pos)).reshape(1, 1)


def kernel(user_table, item_table, user_id, item_id):
    uid = user_id.reshape(_B)
    iidT = item_id.T.reshape(_NPAIR * _B)
    cat = jnp.concatenate([user_table, item_table], axis=1)

    sc = pl.kernel(
        _sc_predictions_kernel,
        out_type=jax.ShapeDtypeStruct((_NW * _NCHUNK * _NPAIR * _C,),
                                      jnp.float32),
        mesh=plsc.VectorSubcoreMesh(core_axis_name="c", subcore_axis_name="s"),
        compiler_params=pltpu.CompilerParams(
            needs_layout_passes=False, use_tc_tiling_on_sc=True),
        scratch_types=[
            pltpu.VMEM((_C,), jnp.int32),
            pltpu.VMEM((_NPAIR * _C,), jnp.int32),
            pltpu.VMEM((_C, _WIDE), jnp.float32),
            pltpu.VMEM((_NPAIR * _C, _WIDE), jnp.float32),
            pltpu.VMEM((_NPAIR * _C,), jnp.float32),
            pltpu.SemaphoreType.DMA,
        ],
    )
    preds = sc(cat, uid, iidT)
    preds = preds.reshape(_NW * _NCHUNK, _NPAIR, _C)

    loss = pl.pallas_call(
        _tc_loss_kernel,
        out_shape=jax.ShapeDtypeStruct((1, 1), jnp.float32),
    )(preds)
    return loss[0, 0]


# speedup vs baseline: 3.5106x; 1.3092x over previous
import functools

import jax
import jax.numpy as jnp
from jax import lax
from jax.experimental import pallas as pl
from jax.experimental.pallas import tpu as pltpu
from jax.experimental.pallas import tpu_sc as plsc

_B = 16384
_D = 64
_NPAIR = 5
_NW = 32
_PER_W = _B // _NW
_C = 64
_NCHUNK = _PER_W // _C
_G = _C // 16


def _sc_predictions_kernel(user_table, item_table, uid_hbm, iidT_hbm, out_hbm,
                           uids_v, iids_v, u_slab, i_slab, pred_v,
                           sem0, sem1):
    sems = (sem0, sem1)
    wid = lax.axis_index("s") * 2 + lax.axis_index("c")
    lane = lax.iota(jnp.int32, 16)

    pltpu.sync_copy(uid_hbm.at[pl.ds(wid * _PER_W, _PER_W)], uids_v)
    for j in range(_NPAIR):
        pltpu.sync_copy(iidT_hbm.at[pl.ds(j * _B + wid * _PER_W, _PER_W)],
                        iids_v.at[pl.ds(j * _PER_W, _PER_W)])

    def issue(c, buf):
        def issue_body(g, carry):
            b0 = c * _C + g * 16
            uvec = uids_v[pl.ds(b0, 16)]
            ivecs = [iids_v[pl.ds(j * _PER_W + b0, 16)]
                     for j in range(_NPAIR)]
            for p in range(16):
                pltpu.async_copy(user_table.at[pl.ds(uvec[p], 1), :],
                                 u_slab.at[buf, pl.ds(g * 16 + p, 1), :],
                                 sems[buf])
                for j in range(_NPAIR):
                    pltpu.async_copy(
                        item_table.at[pl.ds(ivecs[j][p], 1), :],
                        i_slab.at[buf, pl.ds(j * _C + g * 16 + p, 1), :],
                        sems[buf])
            return carry

        lax.fori_loop(0, _G, issue_body, 0)

    def drain(buf):
        pltpu.make_async_copy(user_table.at[pl.ds(0, _C), :],
                              u_slab.at[buf], sems[buf]).wait()
        pltpu.make_async_copy(item_table.at[pl.ds(0, _NPAIR * _C), :],
                              i_slab.at[buf], sems[buf]).wait()

    def compute(c, buf):
        def group_body(g, carry):
            b0 = g * 16
            res = [jnp.zeros((16,), jnp.float32) for _ in range(_NPAIR)]
            for p in range(16):
                b = b0 + p
                us = [u_slab[buf, b, pl.ds(q * 16, 16)]
                      for q in range(_D // 16)]
                for j in range(_NPAIR):
                    r = j * _C + b
                    prod = us[0] * i_slab[buf, r, pl.ds(0, 16)]
                    for q in range(1, _D // 16):
                        prod = prod + us[q] * i_slab[buf, r,
                                                     pl.ds(q * 16, 16)]
                    s = jnp.sum(prod)
                    res[j] = jnp.where(lane == p, s, res[j])
            for j in range(_NPAIR):
                pred_v[pl.ds(j * _C + b0, 16)] = res[j]
            return carry

        lax.fori_loop(0, _G, group_body, 0)
        pltpu.sync_copy(
            pred_v,
            out_hbm.at[pl.ds((wid * _NCHUNK + c) * _NPAIR * _C, _NPAIR * _C)])

    issue(0, 0)
    for c in range(_NCHUNK):
        buf = c % 2
        if c + 1 < _NCHUNK:
            issue(c + 1, 1 - buf)
        drain(buf)
        compute(c, buf)


def _tc_loss_kernel(pred_ref, out_ref):
    p = pred_ref[...]
    pos = p[:, 0:1, :]
    negs = p[:, 1:_NPAIR, :]
    out_ref[...] = jnp.mean(jax.nn.softplus(negs - pos)).reshape(1, 1)


def kernel(user_table, item_table, user_id, item_id):
    uid = user_id.reshape(_B)
    iidT = item_id.T.reshape(_NPAIR * _B)

    sc = pl.kernel(
        _sc_predictions_kernel,
        out_type=jax.ShapeDtypeStruct((_NW * _NCHUNK * _NPAIR * _C,),
                                      jnp.float32),
        mesh=plsc.VectorSubcoreMesh(core_axis_name="c", subcore_axis_name="s"),
        compiler_params=pltpu.CompilerParams(
            needs_layout_passes=False, use_tc_tiling_on_sc=True),
        scratch_types=[
            pltpu.VMEM((_PER_W,), jnp.int32),
            pltpu.VMEM((_NPAIR * _PER_W,), jnp.int32),
            pltpu.VMEM((2, _C, _D), jnp.float32),
            pltpu.VMEM((2, _NPAIR * _C, _D), jnp.float32),
            pltpu.VMEM((_NPAIR * _C,), jnp.float32),
            pltpu.SemaphoreType.DMA,
            pltpu.SemaphoreType.DMA,
        ],
    )
    preds = sc(user_table, item_table, uid, iidT)
    preds = preds.reshape(_NW * _NCHUNK, _NPAIR, _C)

    loss = pl.pallas_call(
        _tc_loss_kernel,
        out_shape=jax.ShapeDtypeStruct((1, 1), jnp.float32),
    )(preds)
    return loss[0, 0]
